# baseline (speedup 1.0000x reference)
import functools

import jax
import jax.numpy as jnp
from jax import lax
from jax.experimental import pallas as pl
from jax.experimental.pallas import tpu as pltpu
from jax.experimental.pallas import tpu_sc as plsc

N = 10000
E = 320000
H = 128
NW = 32
EPW = E // NW
CH = 128
NFULL = EPW // CH
TAIL = EPW - NFULL * CH
RPT = N // 16
RB = 1000
GRID = N // RB
NPAD = 10240
NEGINF = float("-inf")


def _sc_mesh():
    return plsc.VectorSubcoreMesh(core_axis_name="c", subcore_axis_name="s")


def _he_gather(emb_pad, xi):
    RW = 312
    GC = 312

    @functools.partial(
        pl.kernel,
        out_type=jax.ShapeDtypeStruct((N, 16), jnp.float32),
        mesh=_sc_mesh(),
        compiler_params=pltpu.CompilerParams(use_tc_tiling_on_sc=False),
        scratch_types=[
            pltpu.VMEM((GC,), jnp.int32),
            pltpu.VMEM((GC, 16), jnp.float32),
            pltpu.VMEM((16,), jnp.int32),
            pltpu.VMEM((16, 16), jnp.float32),
            pltpu.SemaphoreType.DMA,
        ],
    )
    def hek(emb_ref, xi_ref, out_ref, idx, rows, idxt, rowst, sem):
        c = lax.axis_index("c")
        s = lax.axis_index("s")
        wid = c * 16 + s
        base = wid * RW
        for i in range(RW // GC):
            off = base + i * GC
            pltpu.sync_copy(xi_ref.at[pl.ds(off, GC)], idx)
            pltpu.async_copy(emb_ref.at[idx], rows, sem).wait()
            pltpu.sync_copy(rows, out_ref.at[pl.ds(off, GC)])

        @pl.when(wid == 0)
        def _():
            pltpu.sync_copy(xi_ref.at[pl.ds(NW * RW, 16)], idxt)
            pltpu.async_copy(emb_ref.at[idxt], rowst, sem).wait()
            pltpu.sync_copy(rowst, out_ref.at[pl.ds(NW * RW, 16)])

    return hek(emb_pad, xi)


def _make_segsum(Dp, ECH):
    NCH = E // ECH
    NBASE = NCH // NW
    NEXTRA = NCH - NBASE * NW
    NMAX = NBASE + 1

    @functools.partial(
        pl.kernel,
        out_type=jax.ShapeDtypeStruct((2, N, Dp), jnp.float32),
        mesh=_sc_mesh(),
        compiler_params=pltpu.CompilerParams(use_tc_tiling_on_sc=False),
        scratch_types=[
            [pltpu.VMEM((2, ECH), jnp.int32) for _ in range(4)],
            [pltpu.VMEM((ECH, Dp), jnp.float32) for _ in range(2)],
            pltpu.VMEM_SHARED((N, Dp), jnp.float32),
            [pltpu.SemaphoreType.DMA for _ in range(2)],
            [pltpu.SemaphoreType.DMA for _ in range(2)],
            [pltpu.SemaphoreType.DMA for _ in range(4)],
        ],
    )
    def seg(tab_ref, il_ref, out_ref,
            ibuf, rows, acc, gsem, ssem, isem):
        c = lax.axis_index("c")
        s = lax.axis_index("s")
        wid = c * 16 + s
        nu = NBASE + jnp.where(wid < NEXTRA, 1, 0)

        def zrow(i, carry):
            offs = list(range(0, Dp - 15, 16))
            if Dp % 16:
                offs.append(Dp - 16)
            for kk in offs:
                rows[0][i, pl.ds(kk, 16)] = jnp.zeros((16,), jnp.float32)
            return carry
        lax.fori_loop(0, min(ECH, RPT), zrow, 0)
        zbase = s * RPT
        ZC = min(ECH, RPT)
        for i in range(RPT // ZC):
            pltpu.sync_copy(rows[0].at[pl.ds(0, ZC)] if ZC < ECH else rows[0],
                            acc.at[pl.ds(zbase + i * ZC, ZC)])
        rem = RPT - (RPT // ZC) * ZC
        if rem:
            pltpu.sync_copy(rows[0].at[pl.ds(0, rem)],
                            acc.at[pl.ds(zbase + (RPT // ZC) * ZC, rem)])
        plsc.subcore_barrier()

        def loadidx(t, q):
            pltpu.async_copy(il_ref.at[wid + t * NW], ibuf[q], isem[q])

        def wait_idx(q):
            pltpu.make_async_copy(il_ref.at[0], ibuf[q], isem[q]).wait()

        def gather(q, r):
            pltpu.async_copy(tab_ref.at[ibuf[q].at[0]], rows[r], gsem[r])

        def wait_gather(r):
            pltpu.make_async_copy(tab_ref.at[ibuf[0].at[0]], rows[r],
                                  gsem[r]).wait()

        def scatter(q, r):
            pltpu.async_copy(rows[r], acc.at[ibuf[q].at[1]], ssem[r],
                             add=True)

        def wait_scatter(r):
            pltpu.make_async_copy(rows[r], acc.at[ibuf[0].at[1]],
                                  ssem[r]).wait()

        def section(t, tdyn=None):
            r, r1 = t % 2, (t + 1) % 2
            q, q1, q2 = t % 4, (t + 1) % 4, (t + 2) % 4
            tt = t if tdyn is None else tdyn

            def grd(cond, fn):
                if tdyn is None and isinstance(cond, bool):
                    if cond:
                        fn()
                else:
                    pl.when(cond)(fn)

            def p_process():
                wait_gather(r)
                scatter(q, r)

            def p_drain():
                wait_scatter(r1)

            def p_load():
                loadidx(tt + 2, q2)

            def p_launch():
                wait_idx(q1)
                gather(q1, r1)

            if tdyn is None:
                p_process()
                if t >= 1:
                    p_drain()
                p_load()
                p_launch()
            else:
                grd(tt < nu, p_process)
                grd(tt - 1 < nu, p_drain)
                grd(tt + 2 < nu, p_load)
                grd(tt + 1 < nu, p_launch)

        loadidx(0, 0)
        loadidx(1, 1)
        wait_idx(0)
        gather(0, 0)
        for t in range(4):
            section(t)

        STEADY = (NBASE - 6) // 4
        def body(i, carry):
            for sct in range(4):
                r, r1 = sct % 2, (sct + 1) % 2
                q, q1, q2 = sct % 4, (sct + 1) % 4, (sct + 2) % 4
                t = 4 * i + sct
                wait_gather(r)
                scatter(q, r)
                wait_scatter(r1)
                loadidx(t + 2, q2)
                wait_idx(q1)
                gather(q1, r1)
            return carry
        lax.fori_loop(1, STEADY + 1, body, 0)

        for t in range(4 * (STEADY + 1), NMAX + 1):
            section(t % 4 + 4, tdyn=t)

        plsc.subcore_barrier()
        pltpu.sync_copy(acc.at[pl.ds(s * RPT, RPT)],
                        out_ref.at[c, pl.ds(s * RPT, RPT)])

    return seg


_segsum16 = _make_segsum(16, 512)
_segsum136 = _make_segsum(136, 160)


def _layer_common(Din, k, parts_ref, f_ref, wl_ref, bl_ref, wr_ref, w_ref):
    p = parts_ref[0] + parts_ref[1]
    deg = jnp.maximum(p[:, Din:Din + 1], 1.0)
    mean = p[:, :Din] / deg
    xin = f_ref[:, :Din]
    h = mean @ wl_ref[...] + bl_ref[...] + xin @ wr_ref[...]
    h = jnp.maximum(h, 0.0)
    w = w_ref[...]
    wt = jnp.transpose(w)
    ut = lax.dot_general(wt, h, (((1,), (1,)), ((), ())),
                         preferred_element_type=jnp.float32)
    ut = ut * lax.rsqrt(jnp.sum(w * w))
    keeprow = jnp.transpose(f_ref[:, Din:Din + 1])
    ueff = jnp.where(keeprow > 0, ut, NEGINF)
    b = lax.bitcast_convert_type(ueff, jnp.uint32)
    keys = jnp.where(ueff < 0, ~b, b | jnp.uint32(0x80000000))

    def it(i, prefix):
        shift = jnp.uint32(31) - i.astype(jnp.uint32)
        cand = prefix | jnp.left_shift(jnp.uint32(1), shift)
        cnt = jnp.sum((keys >= cand).astype(jnp.int32))
        return jnp.where(cnt >= k, cand, prefix)

    thr = lax.fori_loop(0, 32, it, jnp.uint32(0))
    keepn = (keys >= thr).astype(jnp.float32)
    grow = jnp.tanh(ut) * keepn
    hm = h * jnp.transpose(grow)
    kp = jnp.transpose(keepn)
    mx = jnp.max(jnp.where(kp > 0, hm, NEGINF), axis=0, keepdims=True)
    sm = jnp.sum(hm, axis=0, keepdims=True) * (1.0 / k)
    z = jnp.concatenate([mx, sm], axis=1)
    return hm, kp, z


def _layer_body(Din, k, parts_ref, f_ref, wl_ref, bl_ref, wr_ref, w_ref,
                fout_ref, z_ref):
    hm, kp, z = _layer_common(Din, k, parts_ref, f_ref, wl_ref, bl_ref,
                              wr_ref, w_ref)
    fout_ref[...] = jnp.concatenate(
        [hm, kp, jnp.zeros((N, 7), jnp.float32)], axis=1)
    z_ref[...] = z


def _layer(parts, F, Wl, bl, Wr, w, Din, k):
    return pl.pallas_call(
        functools.partial(_layer_body, Din, k),
        out_shape=[
            jax.ShapeDtypeStruct((N, 136), jnp.float32),
            jax.ShapeDtypeStruct((1, 2 * H), jnp.float32),
        ],
    )(parts, F, Wl, bl, Wr, w)


def _layer3_body(Din, k, parts_ref, f_ref, wl_ref, bl_ref, wr_ref, w_ref,
                 z1_ref, z2_ref, w1_ref, b1_ref, w2_ref, b2_ref,
                 w3_ref, b3_ref, out_ref):
    _, _, z3 = _layer_common(Din, k, parts_ref, f_ref, wl_ref, bl_ref,
                             wr_ref, w_ref)
    z = z1_ref[...] + z2_ref[...] + z3
    z = jnp.maximum(z @ w1_ref[...] + b1_ref[...], 0.0)
    z = jnp.maximum(z @ w2_ref[...] + b2_ref[...], 0.0)
    z = z @ w3_ref[...] + b3_ref[...]
    out_ref[...] = 1.0 / (1.0 + jnp.exp(-z))


def _layer3(parts, F, Wl, bl, Wr, w, z1, z2, mlp, Din, k):
    W1, b1, W2, b2, W3, b3 = mlp
    return pl.pallas_call(
        functools.partial(_layer3_body, Din, k),
        out_shape=jax.ShapeDtypeStruct((1, 1), jnp.float32),
    )(parts, F, Wl, bl, Wr, w, z1, z2, W1, b1, W2, b2, W3, b3)


def kernel(x, edge_index, batch, emb,
           conv1_Wl, conv1_bl, conv1_Wr, pool1_w,
           conv2_Wl, conv2_bl, conv2_Wr, pool2_w,
           conv3_Wl, conv3_bl, conv3_Wr, pool3_w,
           lin1_W, lin1_b, lin2_W, lin2_b, lin3_W, lin3_b):
    V = emb.shape[0]
    xi = x[:, 0]
    src1, dst1 = jnp.asarray(edge_index[0]), jnp.asarray(edge_index[1])
    il160 = jnp.concatenate([src1.reshape(E // 160, 1, 160),
                             dst1.reshape(E // 160, 1, 160)], axis=1)
    il512 = jnp.concatenate([src1.reshape(E // 512, 1, 512),
                             dst1.reshape(E // 512, 1, 512)], axis=1)

    emb_pad = jnp.concatenate(
        [emb, jnp.ones((V, 1), jnp.float32), jnp.zeros((V, 6), jnp.float32)],
        axis=1)

    F = _he_gather(emb_pad, xi)
    parts = _segsum16(F, il512)
    F, z1 = _layer(parts, F, conv1_Wl, conv1_bl[None, :], conv1_Wr,
                   pool1_w[:, None], 9, 8000)
    parts = _segsum136(F, il160)
    F, z2 = _layer(parts, F, conv2_Wl, conv2_bl[None, :], conv2_Wr,
                   pool2_w[:, None], 128, 6400)
    parts = _segsum136(F, il160)
    out = _layer3(parts, F, conv3_Wl, conv3_bl[None, :], conv3_Wr,
                  pool3_w[:, None], z1, z2,
                  (lin1_W, lin1_b[None, :], lin2_W, lin2_b[None, :],
                   lin3_W, lin3_b[None, :]), 128, 5120)
    return out[:, 0]


# speedup vs baseline: 1.0192x; 1.0192x over previous
import functools

import jax
import jax.numpy as jnp
from jax import lax
from jax.experimental import pallas as pl
from jax.experimental.pallas import tpu as pltpu
from jax.experimental.pallas import tpu_sc as plsc

N = 10000
E = 320000
H = 128
NW = 32
EPW = E // NW
CH = 128
NFULL = EPW // CH
TAIL = EPW - NFULL * CH
RPT = N // 16
RB = 1000
GRID = N // RB
NPAD = 10240
NEGINF = float("-inf")


def _sc_mesh():
    return plsc.VectorSubcoreMesh(core_axis_name="c", subcore_axis_name="s")


def _he_gather(emb_pad, xi):
    RW = 312
    GC = 312

    @functools.partial(
        pl.kernel,
        out_type=jax.ShapeDtypeStruct((N, 16), jnp.float32),
        mesh=_sc_mesh(),
        compiler_params=pltpu.CompilerParams(use_tc_tiling_on_sc=False),
        scratch_types=[
            pltpu.VMEM((GC,), jnp.int32),
            pltpu.VMEM((GC, 16), jnp.float32),
            pltpu.VMEM((16,), jnp.int32),
            pltpu.VMEM((16, 16), jnp.float32),
            pltpu.SemaphoreType.DMA,
        ],
    )
    def hek(emb_ref, xi_ref, out_ref, idx, rows, idxt, rowst, sem):
        c = lax.axis_index("c")
        s = lax.axis_index("s")
        wid = c * 16 + s
        base = wid * RW
        for i in range(RW // GC):
            off = base + i * GC
            pltpu.sync_copy(xi_ref.at[pl.ds(off, GC)], idx)
            pltpu.async_copy(emb_ref.at[idx], rows, sem).wait()
            pltpu.sync_copy(rows, out_ref.at[pl.ds(off, GC)])

        @pl.when(wid == 0)
        def _():
            pltpu.sync_copy(xi_ref.at[pl.ds(NW * RW, 16)], idxt)
            pltpu.async_copy(emb_ref.at[idxt], rowst, sem).wait()
            pltpu.sync_copy(rowst, out_ref.at[pl.ds(NW * RW, 16)])

    return hek(emb_pad, xi)


def _make_segsum(Dp, ECH):
    NCH = E // ECH
    NBASE = NCH // NW
    NEXTRA = NCH - NBASE * NW
    NMAX = NBASE + 1

    @functools.partial(
        pl.kernel,
        out_type=jax.ShapeDtypeStruct((2, N, Dp), jnp.float32),
        mesh=_sc_mesh(),
        compiler_params=pltpu.CompilerParams(use_tc_tiling_on_sc=False),
        scratch_types=[
            [pltpu.VMEM((2, ECH), jnp.int32) for _ in range(4)],
            [pltpu.VMEM((ECH, Dp), jnp.float32) for _ in range(2)],
            pltpu.VMEM_SHARED((N, Dp), jnp.float32),
            [pltpu.SemaphoreType.DMA for _ in range(2)],
            [pltpu.SemaphoreType.DMA for _ in range(2)],
            [pltpu.SemaphoreType.DMA for _ in range(4)],
        ],
    )
    def seg(tab_ref, il_ref, out_ref,
            ibuf, rows, acc, gsem, ssem, isem):
        c = lax.axis_index("c")
        s = lax.axis_index("s")
        wid = c * 16 + s
        nu = NBASE + jnp.where(wid < NEXTRA, 1, 0)

        def zrow(i, carry):
            for kk in range(Dp // 16):
                rows[0][i, pl.ds(kk * 16, 16)] = jnp.zeros((16,), jnp.float32)
            return carry
        lax.fori_loop(0, min(ECH, RPT), zrow, 0)
        zbase = s * RPT
        ZC = min(ECH, RPT)
        for i in range(RPT // ZC):
            pltpu.sync_copy(rows[0].at[pl.ds(0, ZC)] if ZC < ECH else rows[0],
                            acc.at[pl.ds(zbase + i * ZC, ZC)])
        rem = RPT - (RPT // ZC) * ZC
        if rem:
            pltpu.sync_copy(rows[0].at[pl.ds(0, rem)],
                            acc.at[pl.ds(zbase + (RPT // ZC) * ZC, rem)])
        plsc.subcore_barrier()

        def loadidx(t, q):
            pltpu.async_copy(il_ref.at[wid + t * NW], ibuf[q], isem[q])

        def wait_idx(q):
            pltpu.make_async_copy(il_ref.at[0], ibuf[q], isem[q]).wait()

        def gather(q, r):
            pltpu.async_copy(tab_ref.at[ibuf[q].at[0]], rows[r], gsem[r])

        def wait_gather(r):
            pltpu.make_async_copy(tab_ref.at[ibuf[0].at[0]], rows[r],
                                  gsem[r]).wait()

        def scatter(q, r):
            pltpu.async_copy(rows[r], acc.at[ibuf[q].at[1]], ssem[r],
                             add=True)

        def wait_scatter(r):
            pltpu.make_async_copy(rows[r], acc.at[ibuf[0].at[1]],
                                  ssem[r]).wait()

        def section(t, tdyn=None):
            r, r1 = t % 2, (t + 1) % 2
            q, q1, q2 = t % 4, (t + 1) % 4, (t + 2) % 4
            tt = t if tdyn is None else tdyn

            def grd(cond, fn):
                if tdyn is None and isinstance(cond, bool):
                    if cond:
                        fn()
                else:
                    pl.when(cond)(fn)

            def p_process():
                wait_gather(r)
                scatter(q, r)

            def p_drain():
                wait_scatter(r1)

            def p_load():
                loadidx(tt + 2, q2)

            def p_launch():
                wait_idx(q1)
                gather(q1, r1)

            if tdyn is None:
                p_process()
                if t >= 1:
                    p_drain()
                p_load()
                p_launch()
            else:
                grd(tt < nu, p_process)
                grd(tt - 1 < nu, p_drain)
                grd(tt + 2 < nu, p_load)
                grd(tt + 1 < nu, p_launch)

        loadidx(0, 0)
        loadidx(1, 1)
        wait_idx(0)
        gather(0, 0)
        for t in range(4):
            section(t)

        STEADY = (NBASE - 6) // 4
        def body(i, carry):
            for sct in range(4):
                r, r1 = sct % 2, (sct + 1) % 2
                q, q1, q2 = sct % 4, (sct + 1) % 4, (sct + 2) % 4
                t = 4 * i + sct
                wait_gather(r)
                scatter(q, r)
                wait_scatter(r1)
                loadidx(t + 2, q2)
                wait_idx(q1)
                gather(q1, r1)
            return carry
        lax.fori_loop(1, STEADY + 1, body, 0)

        for t in range(4 * (STEADY + 1), NMAX + 1):
            section(t % 4 + 4, tdyn=t)

        plsc.subcore_barrier()
        pltpu.sync_copy(acc.at[pl.ds(s * RPT, RPT)],
                        out_ref.at[c, pl.ds(s * RPT, RPT)])

    return seg


_segsum16 = _make_segsum(16, 512)
_segsum144 = _make_segsum(144, 128)


def _layer_common(Din, k, parts_ref, f_ref, wl_ref, bl_ref, wr_ref, w_ref):
    p = parts_ref[0] + parts_ref[1]
    deg = jnp.maximum(p[:, Din:Din + 1], 1.0)
    mean = p[:, :Din] / deg
    xin = f_ref[:, :Din]
    h = mean @ wl_ref[...] + bl_ref[...] + xin @ wr_ref[...]
    h = jnp.maximum(h, 0.0)
    w = w_ref[...]
    wt = jnp.transpose(w)
    ut = lax.dot_general(wt, h, (((1,), (1,)), ((), ())),
                         preferred_element_type=jnp.float32)
    ut = ut * lax.rsqrt(jnp.sum(w * w))
    keeprow = jnp.transpose(f_ref[:, Din:Din + 1])
    ueff = jnp.where(keeprow > 0, ut, NEGINF)
    b = lax.bitcast_convert_type(ueff, jnp.uint32)
    keys = jnp.where(ueff < 0, ~b, b | jnp.uint32(0x80000000))

    def it(i, prefix):
        shift = jnp.uint32(31) - i.astype(jnp.uint32)
        cand = prefix | jnp.left_shift(jnp.uint32(1), shift)
        cnt = jnp.sum((keys >= cand).astype(jnp.int32))
        return jnp.where(cnt >= k, cand, prefix)

    thr = lax.fori_loop(0, 32, it, jnp.uint32(0))
    keepn = (keys >= thr).astype(jnp.float32)
    grow = jnp.tanh(ut) * keepn
    hm = h * jnp.transpose(grow)
    kp = jnp.transpose(keepn)
    mx = jnp.max(jnp.where(kp > 0, hm, NEGINF), axis=0, keepdims=True)
    sm = jnp.sum(hm, axis=0, keepdims=True) * (1.0 / k)
    z = jnp.concatenate([mx, sm], axis=1)
    return hm, kp, z


def _layer_body(Din, k, parts_ref, f_ref, wl_ref, bl_ref, wr_ref, w_ref,
                fout_ref, z_ref):
    hm, kp, z = _layer_common(Din, k, parts_ref, f_ref, wl_ref, bl_ref,
                              wr_ref, w_ref)
    fout_ref[...] = jnp.concatenate(
        [hm, kp, jnp.zeros((N, 15), jnp.float32)], axis=1)
    z_ref[...] = z


def _layer(parts, F, Wl, bl, Wr, w, Din, k):
    return pl.pallas_call(
        functools.partial(_layer_body, Din, k),
        out_shape=[
            jax.ShapeDtypeStruct((N, 144), jnp.float32),
            jax.ShapeDtypeStruct((1, 2 * H), jnp.float32),
        ],
    )(parts, F, Wl, bl, Wr, w)


def _layer3_body(Din, k, parts_ref, f_ref, wl_ref, bl_ref, wr_ref, w_ref,
                 z1_ref, z2_ref, w1_ref, b1_ref, w2_ref, b2_ref,
                 w3_ref, b3_ref, out_ref):
    _, _, z3 = _layer_common(Din, k, parts_ref, f_ref, wl_ref, bl_ref,
                             wr_ref, w_ref)
    z = z1_ref[...] + z2_ref[...] + z3
    z = jnp.maximum(z @ w1_ref[...] + b1_ref[...], 0.0)
    z = jnp.maximum(z @ w2_ref[...] + b2_ref[...], 0.0)
    z = z @ w3_ref[...] + b3_ref[...]
    out_ref[...] = 1.0 / (1.0 + jnp.exp(-z))


def _layer3(parts, F, Wl, bl, Wr, w, z1, z2, mlp, Din, k):
    W1, b1, W2, b2, W3, b3 = mlp
    return pl.pallas_call(
        functools.partial(_layer3_body, Din, k),
        out_shape=jax.ShapeDtypeStruct((1, 1), jnp.float32),
    )(parts, F, Wl, bl, Wr, w, z1, z2, W1, b1, W2, b2, W3, b3)


def kernel(x, edge_index, batch, emb,
           conv1_Wl, conv1_bl, conv1_Wr, pool1_w,
           conv2_Wl, conv2_bl, conv2_Wr, pool2_w,
           conv3_Wl, conv3_bl, conv3_Wr, pool3_w,
           lin1_W, lin1_b, lin2_W, lin2_b, lin3_W, lin3_b):
    V = emb.shape[0]
    xi = x[:, 0]
    src1, dst1 = jnp.asarray(edge_index[0]), jnp.asarray(edge_index[1])
    il128 = jnp.concatenate([src1.reshape(E // 128, 1, 128),
                             dst1.reshape(E // 128, 1, 128)], axis=1)
    il512 = jnp.concatenate([src1.reshape(E // 512, 1, 512),
                             dst1.reshape(E // 512, 1, 512)], axis=1)

    emb_pad = jnp.concatenate(
        [emb, jnp.ones((V, 1), jnp.float32), jnp.zeros((V, 6), jnp.float32)],
        axis=1)

    F = _he_gather(emb_pad, xi)
    parts = _segsum16(F, il512)
    F, z1 = _layer(parts, F, conv1_Wl, conv1_bl[None, :], conv1_Wr,
                   pool1_w[:, None], 9, 8000)
    parts = _segsum144(F, il128)
    F, z2 = _layer(parts, F, conv2_Wl, conv2_bl[None, :], conv2_Wr,
                   pool2_w[:, None], 128, 6400)
    parts = _segsum144(F, il128)
    out = _layer3(parts, F, conv3_Wl, conv3_bl[None, :], conv3_Wr,
                  pool3_w[:, None], z1, z2,
                  (lin1_W, lin1_b[None, :], lin2_W, lin2_b[None, :],
                   lin3_W, lin3_b[None, :]), 128, 5120)
    return out[:, 0]
